# Initial kernel scaffold; baseline (speedup 1.0000x reference)
import jax
import jax.numpy as jnp
from jax.experimental import pallas as pl


def kernel(input, params):
    raise NotImplementedError("write your pallas kernel here")


# speedup vs baseline: 1.4956x; 1.4956x over previous
import jax
import jax.numpy as jnp
import numpy as np
from jax.experimental import pallas as pl
from jax.experimental.pallas import tpu as pltpu

F32 = jnp.float32
BF16 = jnp.bfloat16


def _dot(a, b):
    return jax.lax.dot_general(a.astype(BF16), b.astype(BF16),
                               (((1,), (0,)), ((), ())),
                               preferred_element_type=F32)


def _dot32(a, b):
    return jax.lax.dot_general(a, b, (((1,), (0,)), ((), ())),
                               preferred_element_type=F32,
                               precision=jax.lax.Precision.HIGHEST)


def _gn_mat(C, HW):
    cg = C // 32
    m = np.zeros((C, C), np.float32)
    for g in range(32):
        m[g * cg:(g + 1) * cg, g * cg:(g + 1) * cg] = 1.0 / (cg * HW)
    return jnp.asarray(m)


def _rb(H):
    return min(16, H)


def _conv_taps(read_slab, w_ref, acc, W, WP, Cout):
    for kh in range(3):
        slab = read_slab(kh)
        for kw in range(3):
            p = _dot(slab, w_ref[kh, kw])
            acc = acc + p.reshape(-1, WP, Cout)[:, kw:kw + W, :]
    return acc


def _make_fused(H, W, C, Cout, pre, has_res, has_nin):
    WP = W + 8
    RB = _rb(H)
    nblk = H // RB

    def kfn(*refs):
        it = iter(refs)
        x_ref = next(it)
        if pre == 'gns':
            g_ref, bt_ref, a_ref = next(it), next(it), next(it)
        w_ref, cb_ref = next(it), next(it)
        r_ref = next(it) if has_res else None
        nw_ref = next(it) if has_nin else None
        o_ref = next(it)
        scr = next(it)

        if pre == 'gns':
            def stats1(ib, s):
                xs = x_ref[0, pl.ds(ib * RB, RB), :, :]
                return s + jnp.sum(jnp.sum(xs, axis=0), axis=0, keepdims=True)

            s1 = jax.lax.fori_loop(0, nblk, stats1, jnp.zeros((1, C), F32),
                                   unroll=False)
            mean = _dot32(s1, a_ref[...])
            mc = mean.reshape(1, 1, C)

            def stats2(ib, q):
                xs = x_ref[0, pl.ds(ib * RB, RB), :, :]
                dv = xs - mc
                return q + jnp.sum(jnp.sum(dv * dv, axis=0), axis=0,
                                   keepdims=True)

            sq = jax.lax.fori_loop(0, nblk, stats2, jnp.zeros((1, C), F32),
                                   unroll=False)
            var = _dot32(sq, a_ref[...])
            rstd = (1.0 / jnp.sqrt(var + 1e-6)).reshape(1, 1, C)
            gg = g_ref[...].reshape(1, 1, C)
            bb = bt_ref[...].reshape(1, 1, C)

        scr[0:1, :, :] = jnp.zeros((1, WP, C), F32)
        scr[H + 1:H + 2, :, :] = jnp.zeros((1, WP, C), F32)
        scr[:, 0:1, :] = jnp.zeros((H + 2, 1, C), F32)
        scr[:, W + 1:WP, :] = jnp.zeros((H + 2, WP - W - 1, C), F32)

        def fill(ib, carry):
            xs = x_ref[0, pl.ds(ib * RB, RB), :, :]
            if pre == 'gns':
                z = ((xs - mc) * rstd) * gg + bb
                z = z * jax.nn.sigmoid(z)
            else:
                z = jnp.where(xs > 0, 1.0, -1.0).astype(F32)
            scr[pl.ds(1 + ib * RB, RB), pl.ds(1, W), :] = z
            return carry

        jax.lax.fori_loop(0, nblk, fill, 0, unroll=False)

        bias = cb_ref[...].reshape(1, 1, Cout)

        def body(ib, carry):
            r0 = ib * RB
            acc = jnp.zeros((RB, W, Cout), F32) + bias
            if has_res:
                r = r_ref[0, pl.ds(r0, RB), :, :]
                if has_nin:
                    r = _dot(r.reshape(RB * W, r.shape[-1]),
                             nw_ref[...]).reshape(RB, W, Cout)
                acc = acc + r
            acc = _conv_taps(
                lambda kh: scr[pl.ds(r0 + kh, RB), :, :].reshape(RB * WP, C),
                w_ref, acc, W, WP, Cout)
            o_ref[0, pl.ds(r0, RB), :, :] = acc
            return carry

        jax.lax.fori_loop(0, nblk, body, 0, unroll=False)

    return kfn


def _sb(shape, index_map):
    return pl.BlockSpec(shape, index_map,
                        pipeline_mode=pl.Buffered(buffer_count=1))


def _fused_conv(x, w4, cb, *, pre, gn=None, res=None, nin_w=None):
    N, H, W, C = x.shape
    Cout = w4.shape[-1]
    WP = W + 8
    kfn = _make_fused(H, W, C, Cout, pre, res is not None, nin_w is not None)

    def full(shape):
        return pl.BlockSpec(shape, lambda n: (0,) * len(shape))

    in_specs = [_sb((1, H, W, C), lambda n: (n, 0, 0, 0))]
    args = [x]
    if pre == 'gns':
        in_specs += [full((1, C)), full((1, C)), full((C, C))]
        args += [gn['g'].reshape(1, C), gn['b'].reshape(1, C),
                 _gn_mat(C, H * W)]
    in_specs += [full((3, 3, C, Cout)), full((1, Cout))]
    args += [w4, cb.reshape(1, Cout)]
    if res is not None:
        Cres = res.shape[-1]
        in_specs.append(_sb((1, H, W, Cres), lambda n: (n, 0, 0, 0)))
        args.append(res)
    if nin_w is not None:
        in_specs.append(full(nin_w.shape))
        args.append(nin_w)
    return pl.pallas_call(
        kfn,
        grid=(N,),
        in_specs=in_specs,
        out_specs=_sb((1, H, W, Cout), lambda n: (n, 0, 0, 0)),
        out_shape=jax.ShapeDtypeStruct((N, H, W, Cout), F32),
        scratch_shapes=[pltpu.VMEM((H + 2, WP, C), F32)],
    )(*args)


def _make_plain(H, W, C, Cout):
    WP = W + 8
    RB = _rb(H)
    nblk = H // RB

    def kfn(xp_ref, w_ref, cb_ref, o_ref):
        bias = cb_ref[...].reshape(1, 1, Cout)

        def body(ib, carry):
            r0 = ib * RB
            acc = jnp.zeros((RB, W, Cout), F32) + bias
            acc = _conv_taps(
                lambda kh: xp_ref[0, pl.ds(r0 + kh, RB), :, :].reshape(RB * WP, C),
                w_ref, acc, W, WP, Cout)
            o_ref[0, pl.ds(r0, RB), :, :] = acc
            return carry

        jax.lax.fori_loop(0, nblk, body, 0, unroll=False)

    return kfn


def _plain_conv(x, w4, cb):
    N, H, W, C = x.shape
    Cout = w4.shape[-1]
    WP = W + 8
    xp = jnp.pad(x, ((0, 0), (1, 1), (1, WP - W - 1), (0, 0)))

    def full(shape):
        return pl.BlockSpec(shape, lambda n: (0,) * len(shape))

    return pl.pallas_call(
        _make_plain(H, W, C, Cout),
        grid=(N,),
        in_specs=[_sb((1, H + 2, WP, C), lambda n: (n, 0, 0, 0)),
                  full((3, 3, C, Cout)), full((1, Cout))],
        out_specs=_sb((1, H, W, Cout), lambda n: (n, 0, 0, 0)),
        out_shape=jax.ShapeDtypeStruct((N, H, W, Cout), F32),
    )(xp, w4, cb.reshape(1, Cout))


def _make_down(Ho, Wo, C, Cout, PH, PW):
    RB = _rb(Ho)
    nblk = Ho // RB

    def kfn(p_ref, w_ref, cb_ref, o_ref):
        bias = cb_ref[...].reshape(1, 1, Cout)

        def body(ib, carry):
            r0 = ib * RB
            acc = jnp.zeros((RB, Wo, Cout), F32) + bias
            for kh in range(3):
                for kw in range(3):
                    pidx = (kh % 2) * 2 + (kw % 2)
                    oh, ow = kh // 2, kw // 2
                    slab = p_ref[0, pidx, pl.ds(oh + r0, RB), :, :]
                    p = _dot(slab.reshape(RB * PW, C), w_ref[kh, kw])
                    acc = acc + p.reshape(RB, PW, Cout)[:, ow:ow + Wo, :]
            o_ref[0, pl.ds(r0, RB), :, :] = acc
            return carry

        jax.lax.fori_loop(0, nblk, body, 0, unroll=False)

    return kfn


def _down_conv(x, w4, cb):
    N, H, W, C = x.shape
    Cout = w4.shape[-1]
    Ho, Wo = H // 2, W // 2
    xp = jnp.pad(x, ((0, 0), (1, 1), (1, 1), (0, 0)))
    planes = jnp.stack([xp[:, a::2, b::2, :] for a in (0, 1) for b in (0, 1)],
                       axis=1)
    PH, PW0 = planes.shape[2], planes.shape[3]
    PW = ((PW0 + 7) // 8) * 8
    planes = jnp.pad(planes, ((0, 0), (0, 0), (0, 0), (0, PW - PW0), (0, 0)))

    def full(shape):
        return pl.BlockSpec(shape, lambda n: (0,) * len(shape))

    return pl.pallas_call(
        _make_down(Ho, Wo, C, Cout, PH, PW),
        grid=(N,),
        in_specs=[_sb((1, 4, PH, PW, C), lambda n: (n, 0, 0, 0, 0)),
                  full((3, 3, C, Cout)), full((1, Cout))],
        out_specs=_sb((1, Ho, Wo, Cout), lambda n: (n, 0, 0, 0)),
        out_shape=jax.ShapeDtypeStruct((N, Ho, Wo, Cout), F32),
    )(planes, w4, cb.reshape(1, Cout))


def _prep_w(p):
    return jnp.transpose(p['w'], (2, 3, 1, 0)), p['b']


def _res_block(x, p):
    w1, b1 = _prep_w(p['conv1'])
    w2, b2 = _prep_w(p['conv2'])
    h = _fused_conv(x, w1, b1, pre='gns', gn=p['norm1'])
    nin_w = None
    cb = b2
    if 'nin' in p:
        nin_w = jnp.transpose(p['nin']['w'][:, :, 0, 0], (1, 0))
        cb = b2 + p['nin']['b']
    return _fused_conv(h, w2, cb, pre='gns', gn=p['norm2'], res=x, nin_w=nin_w)


def _encode(x, enc):
    w, b = _prep_w(enc['conv_in'])
    h = _plain_conv(x, w, b)
    for lvl in enc['down']:
        for blk in lvl['blocks']:
            h = _res_block(h, blk)
        if 'down' in lvl:
            wd, bd = _prep_w(lvl['down'])
            h = _down_conv(h, wd, bd)
    h = _res_block(h, enc['mid1'])
    h = _res_block(h, enc['mid2'])
    w, b = _prep_w(enc['conv_out'])
    return _fused_conv(h, w, b, pre='gns', gn=enc['norm_out'])


def _decode(z, dec):
    w, b = _prep_w(dec['conv_in'])
    h = _fused_conv(z, w, b, pre='sign')
    h = _res_block(h, dec['mid1'])
    h = _res_block(h, dec['mid2'])
    for lvl in dec['up']:
        for blk in lvl['blocks']:
            h = _res_block(h, blk)
        if 'up' in lvl:
            h = jnp.repeat(jnp.repeat(h, 2, axis=1), 2, axis=2)
            wu, bu = _prep_w(lvl['up'])
            h = _plain_conv(h, wu, bu)
    w, b = _prep_w(dec['conv_out'])
    return _fused_conv(h, w, b, pre='gns', gn=dec['norm_out'])


def kernel(input, params):
    x = jnp.transpose(input, (0, 2, 3, 1))
    h = _encode(x, params['enc'])
    d = _decode(h, params['dec'])
    return jnp.transpose(d, (0, 3, 1, 2))


# speedup vs baseline: 1.6027x; 1.0717x over previous
import jax
import jax.numpy as jnp
import numpy as np
from jax.experimental import pallas as pl
from jax.experimental.pallas import tpu as pltpu

F32 = jnp.float32
BF16 = jnp.bfloat16


def _dot(a, b):
    return jax.lax.dot_general(a.astype(BF16), b.astype(BF16),
                               (((1,), (0,)), ((), ())),
                               preferred_element_type=F32)


def _dot32(a, b):
    return jax.lax.dot_general(a, b, (((1,), (0,)), ((), ())),
                               preferred_element_type=F32,
                               precision=jax.lax.Precision.HIGHEST)


def _gn_mat(C, HW):
    cg = C // 32
    m = np.zeros((C, C), np.float32)
    for g in range(32):
        m[g * cg:(g + 1) * cg, g * cg:(g + 1) * cg] = 1.0 / (cg * HW)
    return jnp.asarray(m)


def _rb(H):
    return min(16, H)


def _conv_taps(read_slab, w_ref, acc, W, WP, Cout):
    for kh in range(3):
        slab = read_slab(kh)
        for kw in range(3):
            p = _dot(slab, w_ref[kh, kw])
            acc = acc + p.reshape(-1, WP, Cout)[:, kw:kw + W, :]
    return acc


def _make_fused(H, W, C, Cout, pre, has_res, has_nin):
    WP = W + 8
    RB = _rb(H)
    nblk = H // RB

    def kfn(*refs):
        it = iter(refs)
        x_ref = next(it)
        if pre == 'gns':
            g_ref, bt_ref, a_ref = next(it), next(it), next(it)
        w_ref, cb_ref = next(it), next(it)
        r_ref = next(it) if has_res else None
        nw_ref = next(it) if has_nin else None
        o_ref = next(it)
        scr = next(it)

        if pre == 'gns':
            def stats1(ib, s):
                xs = x_ref[0, pl.ds(ib * RB, RB), :, :]
                return s + jnp.sum(jnp.sum(xs, axis=0), axis=0, keepdims=True)

            s1 = jax.lax.fori_loop(0, nblk, stats1, jnp.zeros((1, C), F32),
                                   unroll=False)
            mean = _dot32(s1, a_ref[...])
            mc = mean.reshape(1, 1, C)

            def stats2(ib, q):
                xs = x_ref[0, pl.ds(ib * RB, RB), :, :]
                dv = xs - mc
                return q + jnp.sum(jnp.sum(dv * dv, axis=0), axis=0,
                                   keepdims=True)

            sq = jax.lax.fori_loop(0, nblk, stats2, jnp.zeros((1, C), F32),
                                   unroll=False)
            var = _dot32(sq, a_ref[...])
            rstd = (1.0 / jnp.sqrt(var + 1e-6)).reshape(1, 1, C)
            gg = g_ref[...].reshape(1, 1, C)
            bb = bt_ref[...].reshape(1, 1, C)

        scr[0:1, :, :] = jnp.zeros((1, WP, C), F32)
        scr[H + 1:H + 2, :, :] = jnp.zeros((1, WP, C), F32)
        scr[:, 0:1, :] = jnp.zeros((H + 2, 1, C), F32)
        scr[:, W + 1:WP, :] = jnp.zeros((H + 2, WP - W - 1, C), F32)

        def fill(ib, carry):
            xs = x_ref[0, pl.ds(ib * RB, RB), :, :]
            if pre == 'gns':
                z = ((xs - mc) * rstd) * gg + bb
                z = z * jax.nn.sigmoid(z)
            elif pre == 'sign':
                z = jnp.where(xs > 0, 1.0, -1.0).astype(F32)
            else:
                z = xs
            scr[pl.ds(1 + ib * RB, RB), pl.ds(1, W), :] = z
            return carry

        jax.lax.fori_loop(0, nblk, fill, 0, unroll=False)

        bias = cb_ref[...].reshape(1, 1, Cout)

        def body(ib, carry):
            r0 = ib * RB
            acc = jnp.zeros((RB, W, Cout), F32) + bias
            if has_res:
                r = r_ref[0, pl.ds(r0, RB), :, :]
                if has_nin:
                    r = _dot(r.reshape(RB * W, r.shape[-1]),
                             nw_ref[...]).reshape(RB, W, Cout)
                acc = acc + r
            acc = _conv_taps(
                lambda kh: scr[pl.ds(r0 + kh, RB), :, :].reshape(RB * WP, C),
                w_ref, acc, W, WP, Cout)
            o_ref[0, pl.ds(r0, RB), :, :] = acc
            return carry

        jax.lax.fori_loop(0, nblk, body, 0, unroll=False)

    return kfn


def _sb(shape, index_map):
    return pl.BlockSpec(shape, index_map,
                        pipeline_mode=pl.Buffered(buffer_count=1))


def _fused_conv(x, w4, cb, *, pre, gn=None, res=None, nin_w=None):
    N, H, W, C = x.shape
    Cout = w4.shape[-1]
    WP = W + 8
    kfn = _make_fused(H, W, C, Cout, pre, res is not None, nin_w is not None)

    def full(shape):
        return pl.BlockSpec(shape, lambda n: (0,) * len(shape))

    in_specs = [_sb((1, H, W, C), lambda n: (n, 0, 0, 0))]
    args = [x]
    if pre == 'gns':
        in_specs += [full((1, C)), full((1, C)), full((C, C))]
        args += [gn['g'].reshape(1, C), gn['b'].reshape(1, C),
                 _gn_mat(C, H * W)]
    in_specs += [full((3, 3, C, Cout)), full((1, Cout))]
    args += [w4, cb.reshape(1, Cout)]
    if res is not None:
        Cres = res.shape[-1]
        in_specs.append(_sb((1, H, W, Cres), lambda n: (n, 0, 0, 0)))
        args.append(res)
    if nin_w is not None:
        in_specs.append(full(nin_w.shape))
        args.append(nin_w)
    return pl.pallas_call(
        kfn,
        grid=(N,),
        in_specs=in_specs,
        out_specs=_sb((1, H, W, Cout), lambda n: (n, 0, 0, 0)),
        out_shape=jax.ShapeDtypeStruct((N, H, W, Cout), F32),
        scratch_shapes=[pltpu.VMEM((H + 2, WP, C), F32)],
    )(*args)


def _make_up(H, W, C, Cout):
    Ho, Wo = 2 * H, 2 * W
    WP = Wo + 8
    RB = _rb(H)
    RBo = _rb(Ho)
    nblk_i = H // RB
    nblk_o = Ho // RBo

    def kfn(x_ref, w_ref, cb_ref, o_ref, scr):
        scr[0:1, :, :] = jnp.zeros((1, WP, C), F32)
        scr[Ho + 1:Ho + 2, :, :] = jnp.zeros((1, WP, C), F32)
        scr[:, 0:1, :] = jnp.zeros((Ho + 2, 1, C), F32)
        scr[:, Wo + 1:WP, :] = jnp.zeros((Ho + 2, WP - Wo - 1, C), F32)

        def fill(ib, carry):
            xs = x_ref[0, pl.ds(ib * RB, RB), :, :]
            z = jnp.repeat(jnp.repeat(xs, 2, axis=0), 2, axis=1)
            scr[pl.ds(1 + 2 * ib * RB, 2 * RB), pl.ds(1, Wo), :] = z
            return carry

        jax.lax.fori_loop(0, nblk_i, fill, 0, unroll=False)

        bias = cb_ref[...].reshape(1, 1, Cout)

        def body(ib, carry):
            r0 = ib * RBo
            acc = jnp.zeros((RBo, Wo, Cout), F32) + bias
            acc = _conv_taps(
                lambda kh: scr[pl.ds(r0 + kh, RBo), :, :].reshape(RBo * WP, C),
                w_ref, acc, Wo, WP, Cout)
            o_ref[0, pl.ds(r0, RBo), :, :] = acc
            return carry

        jax.lax.fori_loop(0, nblk_o, body, 0, unroll=False)

    return kfn


def _up_conv(x, w4, cb):
    N, H, W, C = x.shape
    Cout = w4.shape[-1]
    WP = 2 * W + 8

    def full(shape):
        return pl.BlockSpec(shape, lambda n: (0,) * len(shape))

    return pl.pallas_call(
        _make_up(H, W, C, Cout),
        grid=(N,),
        in_specs=[_sb((1, H, W, C), lambda n: (n, 0, 0, 0)),
                  full((3, 3, C, Cout)), full((1, Cout))],
        out_specs=_sb((1, 2 * H, 2 * W, Cout), lambda n: (n, 0, 0, 0)),
        out_shape=jax.ShapeDtypeStruct((N, 2 * H, 2 * W, Cout), F32),
        scratch_shapes=[pltpu.VMEM((2 * H + 2, WP, C), F32)],
    )(x, w4, cb.reshape(1, Cout))


def _make_plain(H, W, C, Cout):
    WP = W + 8
    RB = _rb(H)
    nblk = H // RB

    def kfn(xp_ref, w_ref, cb_ref, o_ref):
        bias = cb_ref[...].reshape(1, 1, Cout)

        def body(ib, carry):
            r0 = ib * RB
            acc = jnp.zeros((RB, W, Cout), F32) + bias
            acc = _conv_taps(
                lambda kh: xp_ref[0, pl.ds(r0 + kh, RB), :, :].reshape(RB * WP, C),
                w_ref, acc, W, WP, Cout)
            o_ref[0, pl.ds(r0, RB), :, :] = acc
            return carry

        jax.lax.fori_loop(0, nblk, body, 0, unroll=False)

    return kfn


def _plain_conv(x, w4, cb):
    N, H, W, C = x.shape
    Cout = w4.shape[-1]
    WP = W + 8
    xp = jnp.pad(x, ((0, 0), (1, 1), (1, WP - W - 1), (0, 0)))

    def full(shape):
        return pl.BlockSpec(shape, lambda n: (0,) * len(shape))

    return pl.pallas_call(
        _make_plain(H, W, C, Cout),
        grid=(N,),
        in_specs=[_sb((1, H + 2, WP, C), lambda n: (n, 0, 0, 0)),
                  full((3, 3, C, Cout)), full((1, Cout))],
        out_specs=_sb((1, H, W, Cout), lambda n: (n, 0, 0, 0)),
        out_shape=jax.ShapeDtypeStruct((N, H, W, Cout), F32),
    )(xp, w4, cb.reshape(1, Cout))


def _make_down(Ho, Wo, C, Cout, PH, PW):
    RB = _rb(Ho)
    nblk = Ho // RB

    def kfn(p_ref, w_ref, cb_ref, o_ref):
        bias = cb_ref[...].reshape(1, 1, Cout)

        def body(ib, carry):
            r0 = ib * RB
            acc = jnp.zeros((RB, Wo, Cout), F32) + bias
            for kh in range(3):
                for kw in range(3):
                    pidx = (kh % 2) * 2 + (kw % 2)
                    oh, ow = kh // 2, kw // 2
                    slab = p_ref[0, pidx, pl.ds(oh + r0, RB), :, :]
                    p = _dot(slab.reshape(RB * PW, C), w_ref[kh, kw])
                    acc = acc + p.reshape(RB, PW, Cout)[:, ow:ow + Wo, :]
            o_ref[0, pl.ds(r0, RB), :, :] = acc
            return carry

        jax.lax.fori_loop(0, nblk, body, 0, unroll=False)

    return kfn


def _down_conv(x, w4, cb):
    N, H, W, C = x.shape
    Cout = w4.shape[-1]
    Ho, Wo = H // 2, W // 2
    xp = jnp.pad(x, ((0, 0), (1, 1), (1, 1), (0, 0)))
    planes = jnp.stack([xp[:, a::2, b::2, :] for a in (0, 1) for b in (0, 1)],
                       axis=1)
    PH, PW0 = planes.shape[2], planes.shape[3]
    PW = ((PW0 + 7) // 8) * 8
    planes = jnp.pad(planes, ((0, 0), (0, 0), (0, 0), (0, PW - PW0), (0, 0)))

    def full(shape):
        return pl.BlockSpec(shape, lambda n: (0,) * len(shape))

    return pl.pallas_call(
        _make_down(Ho, Wo, C, Cout, PH, PW),
        grid=(N,),
        in_specs=[_sb((1, 4, PH, PW, C), lambda n: (n, 0, 0, 0, 0)),
                  full((3, 3, C, Cout)), full((1, Cout))],
        out_specs=_sb((1, Ho, Wo, Cout), lambda n: (n, 0, 0, 0)),
        out_shape=jax.ShapeDtypeStruct((N, Ho, Wo, Cout), F32),
    )(planes, w4, cb.reshape(1, Cout))


def _prep_w(p):
    return jnp.transpose(p['w'], (2, 3, 1, 0)), p['b']


def _res_block(x, p):
    w1, b1 = _prep_w(p['conv1'])
    w2, b2 = _prep_w(p['conv2'])
    h = _fused_conv(x, w1, b1, pre='gns', gn=p['norm1'])
    nin_w = None
    cb = b2
    if 'nin' in p:
        nin_w = jnp.transpose(p['nin']['w'][:, :, 0, 0], (1, 0))
        cb = b2 + p['nin']['b']
    return _fused_conv(h, w2, cb, pre='gns', gn=p['norm2'], res=x, nin_w=nin_w)


def _encode(x, enc):
    w, b = _prep_w(enc['conv_in'])
    h = _fused_conv(x, w, b, pre='copy')
    for lvl in enc['down']:
        for blk in lvl['blocks']:
            h = _res_block(h, blk)
        if 'down' in lvl:
            wd, bd = _prep_w(lvl['down'])
            h = _down_conv(h, wd, bd)
    h = _res_block(h, enc['mid1'])
    h = _res_block(h, enc['mid2'])
    w, b = _prep_w(enc['conv_out'])
    return _fused_conv(h, w, b, pre='gns', gn=enc['norm_out'])


def _decode(z, dec):
    w, b = _prep_w(dec['conv_in'])
    h = _fused_conv(z, w, b, pre='sign')
    h = _res_block(h, dec['mid1'])
    h = _res_block(h, dec['mid2'])
    for lvl in dec['up']:
        for blk in lvl['blocks']:
            h = _res_block(h, blk)
        if 'up' in lvl:
            wu, bu = _prep_w(lvl['up'])
            h = _up_conv(h, wu, bu)
    w, b = _prep_w(dec['conv_out'])
    return _fused_conv(h, w, b, pre='gns', gn=dec['norm_out'])


def kernel(input, params):
    x = jnp.transpose(input, (0, 2, 3, 1))
    h = _encode(x, params['enc'])
    d = _decode(h, params['dec'])
    return jnp.transpose(d, (0, 3, 1, 2))


# speedup vs baseline: 1.6876x; 1.0529x over previous
import jax
import jax.numpy as jnp
import numpy as np
from jax.experimental import pallas as pl
from jax.experimental.pallas import tpu as pltpu

F32 = jnp.float32
BF16 = jnp.bfloat16


def _dot(a, b):
    return jax.lax.dot_general(a.astype(BF16), b.astype(BF16),
                               (((1,), (0,)), ((), ())),
                               preferred_element_type=F32)


def _dot32(a, b):
    return jax.lax.dot_general(a, b, (((1,), (0,)), ((), ())),
                               preferred_element_type=F32,
                               precision=jax.lax.Precision.HIGHEST)


def _gn_mat(C, HW):
    cg = C // 32
    m = np.zeros((C, C), np.float32)
    for g in range(32):
        m[g * cg:(g + 1) * cg, g * cg:(g + 1) * cg] = 1.0 / (cg * HW)
    return jnp.asarray(m)


def _rb(H):
    return min(16, H)


def _conv_taps(read_slab, w_ref, acc, W, WP, Cout):
    for kh in range(3):
        slab = read_slab(kh)
        for kw in range(3):
            p = _dot(slab, w_ref[kh, kw])
            acc = acc + p.reshape(-1, WP, Cout)[:, kw:kw + W, :]
    return acc


def _make_fused(H, W, C, Cout, pre, has_res, has_nin):
    WP = W + 8
    RB = _rb(H)
    nblk = H // RB

    def kfn(*refs):
        it = iter(refs)
        x_ref = next(it)
        if pre == 'gns':
            g_ref, bt_ref, a_ref = next(it), next(it), next(it)
        w_ref, cb_ref = next(it), next(it)
        r_ref = next(it) if has_res else None
        nw_ref = next(it) if has_nin else None
        o_ref = next(it)
        scr = next(it)

        if pre == 'gns':
            def stats1(ib, s):
                xs = x_ref[0, pl.ds(ib * RB, RB), :, :]
                return s + jnp.sum(jnp.sum(xs, axis=0), axis=0, keepdims=True)

            s1 = jax.lax.fori_loop(0, nblk, stats1, jnp.zeros((1, C), F32),
                                   unroll=False)
            mean = _dot32(s1, a_ref[...])
            mc = mean.reshape(1, 1, C)

            def stats2(ib, q):
                xs = x_ref[0, pl.ds(ib * RB, RB), :, :]
                dv = xs - mc
                return q + jnp.sum(jnp.sum(dv * dv, axis=0), axis=0,
                                   keepdims=True)

            sq = jax.lax.fori_loop(0, nblk, stats2, jnp.zeros((1, C), F32),
                                   unroll=False)
            var = _dot32(sq, a_ref[...])
            rstd = (1.0 / jnp.sqrt(var + 1e-6)).reshape(1, 1, C)
            gg = g_ref[...].reshape(1, 1, C)
            bb = bt_ref[...].reshape(1, 1, C)

        scr[0:1, :, :] = jnp.zeros((1, WP, C), F32)
        scr[H + 1:H + 2, :, :] = jnp.zeros((1, WP, C), F32)
        scr[:, 0:1, :] = jnp.zeros((H + 2, 1, C), F32)
        scr[:, W + 1:WP, :] = jnp.zeros((H + 2, WP - W - 1, C), F32)

        def fill(ib, carry):
            xs = x_ref[0, pl.ds(ib * RB, RB), :, :]
            if pre == 'gns':
                z = ((xs - mc) * rstd) * gg + bb
                z = z * jax.nn.sigmoid(z)
            elif pre == 'sign':
                z = jnp.where(xs > 0, 1.0, -1.0).astype(F32)
            else:
                z = xs
            scr[pl.ds(1 + ib * RB, RB), pl.ds(1, W), :] = z
            return carry

        jax.lax.fori_loop(0, nblk, fill, 0, unroll=False)

        bias = cb_ref[...].reshape(1, 1, Cout)

        def body(ib, carry):
            r0 = ib * RB
            acc = jnp.zeros((RB, W, Cout), F32) + bias
            if has_res:
                r = r_ref[0, pl.ds(r0, RB), :, :]
                if has_nin:
                    r = _dot(r.reshape(RB * W, r.shape[-1]),
                             nw_ref[...]).reshape(RB, W, Cout)
                acc = acc + r
            acc = _conv_taps(
                lambda kh: scr[pl.ds(r0 + kh, RB), :, :].reshape(RB * WP, C),
                w_ref, acc, W, WP, Cout)
            o_ref[0, pl.ds(r0, RB), :, :] = acc
            return carry

        jax.lax.fori_loop(0, nblk, body, 0, unroll=False)

    return kfn


def _sb(shape, index_map):
    return pl.BlockSpec(shape, index_map,
                        pipeline_mode=pl.Buffered(buffer_count=1))


def _fused_conv(x, w4, cb, *, pre, gn=None, res=None, nin_w=None):
    N, H, W, C = x.shape
    Cout = w4.shape[-1]
    WP = W + 8
    kfn = _make_fused(H, W, C, Cout, pre, res is not None, nin_w is not None)

    def full(shape):
        return pl.BlockSpec(shape, lambda n: (0,) * len(shape))

    in_specs = [_sb((1, H, W, C), lambda n: (n, 0, 0, 0))]
    args = [x]
    if pre == 'gns':
        in_specs += [full((1, C)), full((1, C)), full((C, C))]
        args += [gn['g'].reshape(1, C), gn['b'].reshape(1, C),
                 _gn_mat(C, H * W)]
    in_specs += [full((3, 3, C, Cout)), full((1, Cout))]
    args += [w4, cb.reshape(1, Cout)]
    if res is not None:
        Cres = res.shape[-1]
        in_specs.append(_sb((1, H, W, Cres), lambda n: (n, 0, 0, 0)))
        args.append(res)
    if nin_w is not None:
        in_specs.append(full(nin_w.shape))
        args.append(nin_w)
    return pl.pallas_call(
        kfn,
        grid=(N,),
        in_specs=in_specs,
        out_specs=_sb((1, H, W, Cout), lambda n: (n, 0, 0, 0)),
        out_shape=jax.ShapeDtypeStruct((N, H, W, Cout), F32),
        scratch_shapes=[pltpu.VMEM((H + 2, WP, C), F32)],
    )(*args)


def _make_up(H, W, C, Cout):
    Ho, Wo = 2 * H, 2 * W
    WP = Wo + 8
    RB = _rb(H)
    RBo = _rb(Ho)
    nblk_i = H // RB
    nblk_o = Ho // RBo

    def kfn(x_ref, w_ref, cb_ref, o_ref, scr):
        scr[0:1, :, :] = jnp.zeros((1, WP, C), F32)
        scr[Ho + 1:Ho + 2, :, :] = jnp.zeros((1, WP, C), F32)
        scr[:, 0:1, :] = jnp.zeros((Ho + 2, 1, C), F32)
        scr[:, Wo + 1:WP, :] = jnp.zeros((Ho + 2, WP - Wo - 1, C), F32)

        def fill(ib, carry):
            xs = x_ref[0, pl.ds(ib * RB, RB), :, :]
            z = jnp.repeat(jnp.repeat(xs, 2, axis=0), 2, axis=1)
            scr[pl.ds(1 + 2 * ib * RB, 2 * RB), pl.ds(1, Wo), :] = z
            return carry

        jax.lax.fori_loop(0, nblk_i, fill, 0, unroll=False)

        bias = cb_ref[...].reshape(1, 1, Cout)

        def body(ib, carry):
            r0 = ib * RBo
            acc = jnp.zeros((RBo, Wo, Cout), F32) + bias
            acc = _conv_taps(
                lambda kh: scr[pl.ds(r0 + kh, RBo), :, :].reshape(RBo * WP, C),
                w_ref, acc, Wo, WP, Cout)
            o_ref[0, pl.ds(r0, RBo), :, :] = acc
            return carry

        jax.lax.fori_loop(0, nblk_o, body, 0, unroll=False)

    return kfn


def _up_conv(x, w4, cb):
    N, H, W, C = x.shape
    Cout = w4.shape[-1]
    WP = 2 * W + 8

    def full(shape):
        return pl.BlockSpec(shape, lambda n: (0,) * len(shape))

    return pl.pallas_call(
        _make_up(H, W, C, Cout),
        grid=(N,),
        in_specs=[_sb((1, H, W, C), lambda n: (n, 0, 0, 0)),
                  full((3, 3, C, Cout)), full((1, Cout))],
        out_specs=_sb((1, 2 * H, 2 * W, Cout), lambda n: (n, 0, 0, 0)),
        out_shape=jax.ShapeDtypeStruct((N, 2 * H, 2 * W, Cout), F32),
        scratch_shapes=[pltpu.VMEM((2 * H + 2, WP, C), F32)],
    )(x, w4, cb.reshape(1, Cout))


def _make_plain(H, W, C, Cout):
    WP = W + 8
    RB = _rb(H)
    nblk = H // RB

    def kfn(xp_ref, w_ref, cb_ref, o_ref):
        bias = cb_ref[...].reshape(1, 1, Cout)

        def body(ib, carry):
            r0 = ib * RB
            acc = jnp.zeros((RB, W, Cout), F32) + bias
            acc = _conv_taps(
                lambda kh: xp_ref[0, pl.ds(r0 + kh, RB), :, :].reshape(RB * WP, C),
                w_ref, acc, W, WP, Cout)
            o_ref[0, pl.ds(r0, RB), :, :] = acc
            return carry

        jax.lax.fori_loop(0, nblk, body, 0, unroll=False)

    return kfn


def _plain_conv(x, w4, cb):
    N, H, W, C = x.shape
    Cout = w4.shape[-1]
    WP = W + 8
    xp = jnp.pad(x, ((0, 0), (1, 1), (1, WP - W - 1), (0, 0)))

    def full(shape):
        return pl.BlockSpec(shape, lambda n: (0,) * len(shape))

    return pl.pallas_call(
        _make_plain(H, W, C, Cout),
        grid=(N,),
        in_specs=[_sb((1, H + 2, WP, C), lambda n: (n, 0, 0, 0)),
                  full((3, 3, C, Cout)), full((1, Cout))],
        out_specs=_sb((1, H, W, Cout), lambda n: (n, 0, 0, 0)),
        out_shape=jax.ShapeDtypeStruct((N, H, W, Cout), F32),
    )(xp, w4, cb.reshape(1, Cout))


def _make_down(Ho, Wo, C, Cout, PH, PW):
    RB = _rb(Ho)
    nblk = Ho // RB

    def kfn(p_ref, w_ref, cb_ref, o_ref):
        bias = cb_ref[...].reshape(1, 1, Cout)

        def body(ib, carry):
            r0 = ib * RB
            acc = jnp.zeros((RB, Wo, Cout), F32) + bias
            for kh in range(3):
                for kw in range(3):
                    pidx = (kh % 2) * 2 + (kw % 2)
                    oh, ow = kh // 2, kw // 2
                    slab = p_ref[0, pidx, pl.ds(oh + r0, RB), :, :]
                    p = _dot(slab.reshape(RB * PW, C), w_ref[kh, kw])
                    acc = acc + p.reshape(RB, PW, Cout)[:, ow:ow + Wo, :]
            o_ref[0, pl.ds(r0, RB), :, :] = acc
            return carry

        jax.lax.fori_loop(0, nblk, body, 0, unroll=False)

    return kfn


def _down_conv(x, w4, cb):
    N, H, W, C = x.shape
    Cout = w4.shape[-1]
    Ho, Wo = H // 2, W // 2
    xp = jnp.pad(x, ((0, 0), (1, 1), (1, 1), (0, 0)))
    planes = jnp.stack([xp[:, a::2, b::2, :] for a in (0, 1) for b in (0, 1)],
                       axis=1)
    PH, PW0 = planes.shape[2], planes.shape[3]
    PW = ((PW0 + 7) // 8) * 8
    planes = jnp.pad(planes, ((0, 0), (0, 0), (0, 0), (0, PW - PW0), (0, 0))).astype(BF16)

    def full(shape):
        return pl.BlockSpec(shape, lambda n: (0,) * len(shape))

    return pl.pallas_call(
        _make_down(Ho, Wo, C, Cout, PH, PW),
        grid=(N,),
        in_specs=[_sb((1, 4, PH, PW, C), lambda n: (n, 0, 0, 0, 0)),
                  full((3, 3, C, Cout)), full((1, Cout))],
        out_specs=_sb((1, Ho, Wo, Cout), lambda n: (n, 0, 0, 0)),
        out_shape=jax.ShapeDtypeStruct((N, Ho, Wo, Cout), F32),
    )(planes, w4, cb.reshape(1, Cout))


def _prep_w(p):
    return jnp.transpose(p['w'], (2, 3, 1, 0)).astype(BF16), p['b']


def _res_block(x, p):
    w1, b1 = _prep_w(p['conv1'])
    w2, b2 = _prep_w(p['conv2'])
    h = _fused_conv(x, w1, b1, pre='gns', gn=p['norm1'])
    nin_w = None
    cb = b2
    if 'nin' in p:
        nin_w = jnp.transpose(p['nin']['w'][:, :, 0, 0], (1, 0)).astype(BF16)
        cb = b2 + p['nin']['b']
    return _fused_conv(h, w2, cb, pre='gns', gn=p['norm2'], res=x, nin_w=nin_w)


def _encode(x, enc):
    w, b = _prep_w(enc['conv_in'])
    h = _fused_conv(x, w, b, pre='copy')
    for lvl in enc['down']:
        for blk in lvl['blocks']:
            h = _res_block(h, blk)
        if 'down' in lvl:
            wd, bd = _prep_w(lvl['down'])
            h = _down_conv(h, wd, bd)
    h = _res_block(h, enc['mid1'])
    h = _res_block(h, enc['mid2'])
    w, b = _prep_w(enc['conv_out'])
    return _fused_conv(h, w, b, pre='gns', gn=enc['norm_out'])


def _decode(z, dec):
    w, b = _prep_w(dec['conv_in'])
    h = _fused_conv(z, w, b, pre='sign')
    h = _res_block(h, dec['mid1'])
    h = _res_block(h, dec['mid2'])
    for lvl in dec['up']:
        for blk in lvl['blocks']:
            h = _res_block(h, blk)
        if 'up' in lvl:
            wu, bu = _prep_w(lvl['up'])
            h = _up_conv(h, wu, bu)
    w, b = _prep_w(dec['conv_out'])
    return _fused_conv(h, w, b, pre='gns', gn=dec['norm_out'])


def kernel(input, params):
    x = jnp.transpose(input, (0, 2, 3, 1))
    h = _encode(x, params['enc'])
    d = _decode(h, params['dec'])
    return jnp.transpose(d, (0, 3, 1, 2))
